# baseline (speedup 1.0000x reference)
import dataclasses
import functools

import jax
import jax.numpy as jnp
from jax import lax
from jax.experimental import pallas as pl
from jax.experimental.pallas import tpu as pltpu
from jax.experimental.pallas import tpu_sc as plsc

N = 10000
E = 160000
D = 256
H = 3
C = 4

NC, NS = 2, 16
NW = NC * NS
NP = 10240
EW = E // NW
EWF = (EW // 16) * 16
BLK = 512
L = 16
EWA = EW + L

_vmesh = plsc.VectorSubcoreMesh(
    core_axis_name="c", subcore_axis_name="s", num_cores=NC, num_subcores=NS
)

_sc_params = pltpu.CompilerParams()
if "needs_layout_passes" in pltpu.CompilerParams.__dataclass_fields__:
    _sc_params = dataclasses.replace(_sc_params, needs_layout_passes=False)


def _tail_mask():
    return lax.iota(jnp.int32, L) < (EW - EWF)


@functools.partial(
    pl.kernel,
    out_type=jax.ShapeDtypeStruct((NW * NP,), jnp.float32),
    mesh=_vmesh,
    scratch_types=[
        pltpu.VMEM((EWA,), jnp.int32),
        pltpu.VMEM((NP,), jnp.float32),
        pltpu.SemaphoreType.DMA,
    ],
    compiler_params=_sc_params,
)
def _sc_degree(dst_hbm, out_hbm, dst_v, deg_v, sem):
    wid = lax.axis_index("c") * NS + lax.axis_index("s")
    cp = pltpu.async_copy(dst_hbm.at[pl.ds(wid * EW, EW)],
                          dst_v.at[pl.ds(0, EW)], sem)

    @pl.loop(0, NP, step=L)
    def _(i):
        deg_v[pl.ds(i, L)] = jnp.zeros((L,), jnp.float32)

    cp.wait()
    ones = jnp.ones((L,), jnp.float32)

    @pl.loop(0, EWF, step=L, unroll=2)
    def _(i):
        plsc.addupdate_scatter(deg_v, [dst_v[pl.ds(i, L)]], ones)

    plsc.addupdate_scatter(
        deg_v, [dst_v[pl.ds(EWF, L)]], ones, mask=_tail_mask()
    )

    pltpu.sync_copy(deg_v, out_hbm.at[pl.ds(wid * NP, NP)])


@functools.partial(
    pl.kernel,
    out_type=jax.ShapeDtypeStruct((NW * H * NP,), jnp.float32),
    mesh=_vmesh,
    scratch_types=[
        pltpu.VMEM((EWA,), jnp.int32),
        pltpu.VMEM((EWA,), jnp.int32),
        pltpu.VMEM((NP,), jnp.float32),
        pltpu.VMEM((NP,), jnp.float32),
        pltpu.VMEM((NP,), jnp.float32),
        pltpu.VMEM((NP,), jnp.float32),
        pltpu.VMEM((NP,), jnp.float32),
        pltpu.VMEM((NP,), jnp.float32),
        pltpu.SemaphoreType.DMA,
    ],
    compiler_params=_sc_params,
)
def _sc_aggregate(src_hbm, dst_hbm, g_hbm, out_hbm,
                  src_v, dst_v, g0, g1, g2, a0, a1, a2, sem):
    wid = lax.axis_index("c") * NS + lax.axis_index("s")
    base = wid * EW
    cps = [
        pltpu.async_copy(src_hbm.at[pl.ds(base, EW)],
                         src_v.at[pl.ds(0, EW)], sem),
        pltpu.async_copy(dst_hbm.at[pl.ds(base, EW)],
                         dst_v.at[pl.ds(0, EW)], sem),
        pltpu.async_copy(g_hbm.at[pl.ds(0, NP)], g0, sem),
        pltpu.async_copy(g_hbm.at[pl.ds(NP, NP)], g1, sem),
        pltpu.async_copy(g_hbm.at[pl.ds(2 * NP, NP)], g2, sem),
    ]

    @pl.loop(0, NP, step=L)
    def _(i):
        z = jnp.zeros((L,), jnp.float32)
        a0[pl.ds(i, L)] = z
        a1[pl.ds(i, L)] = z
        a2[pl.ds(i, L)] = z

    for cp in cps:
        cp.wait()

    def edge(i, mask=None):
        s = src_v[pl.ds(i, L)]
        d = dst_v[pl.ds(i, L)]
        plsc.addupdate_scatter(a0, [d], plsc.load_gather(g0, [s], mask=mask),
                               mask=mask)
        plsc.addupdate_scatter(a1, [d], plsc.load_gather(g1, [s], mask=mask),
                               mask=mask)
        plsc.addupdate_scatter(a2, [d], plsc.load_gather(g2, [s], mask=mask),
                               mask=mask)

    @pl.loop(0, EWF, step=L, unroll=2)
    def _(i):
        edge(i)

    edge(EWF, mask=_tail_mask())

    obase = wid * (H * NP)
    pltpu.sync_copy(a0, out_hbm.at[pl.ds(obase, NP)])
    pltpu.sync_copy(a1, out_hbm.at[pl.ds(obase + NP, NP)])
    pltpu.sync_copy(a2, out_hbm.at[pl.ds(obase + 2 * NP, NP)])


def _tch_body(x_ref, w_ref, hp_ref):
    h = jnp.dot(x_ref[...], w_ref[...],
                preferred_element_type=jnp.float32)
    hp_ref[...] = h.T


def _tch(x, w):
    return pl.pallas_call(
        _tch_body,
        grid=(NP // BLK,),
        in_specs=[
            pl.BlockSpec((BLK, D), lambda i: (i, 0)),
            pl.BlockSpec((D, H), lambda i: (0, 0)),
        ],
        out_specs=pl.BlockSpec((H, BLK), lambda i: (0, i)),
        out_shape=jax.ShapeDtypeStruct((H, NP), jnp.float32),
    )(x, w)


def _tcg_body(hp_ref, degp_ref, g_ref, dinv_ref):
    deg = jnp.sum(degp_ref[...], axis=0, keepdims=True) + 1.0
    dinv = lax.rsqrt(deg)
    g_ref[...] = hp_ref[...] * dinv
    dinv_ref[...] = dinv


def _tcg(hp, degp):
    return pl.pallas_call(
        _tcg_body,
        grid=(NP // BLK,),
        in_specs=[
            pl.BlockSpec((H, BLK), lambda i: (0, i)),
            pl.BlockSpec((NW, BLK), lambda i: (0, i)),
        ],
        out_specs=[
            pl.BlockSpec((H, BLK), lambda i: (0, i)),
            pl.BlockSpec((1, BLK), lambda i: (0, i)),
        ],
        out_shape=[
            jax.ShapeDtypeStruct((H, NP), jnp.float32),
            jax.ShapeDtypeStruct((1, NP), jnp.float32),
        ],
    )(hp, degp)


def _tco_body(accp_ref, g_ref, dinv_ref, bg_ref, wo_ref, bo_ref,
              hr_ref, z_ref):
    acc = jnp.sum(accp_ref[...], axis=0) + g_ref[...]
    hg = acc * dinv_ref[...] + bg_ref[...]
    hr = jnp.maximum(hg, 0.0)
    hr_row = hr.T
    hr_ref[...] = hr_row
    z_ref[...] = jnp.dot(hr_row, wo_ref[...],
                         preferred_element_type=jnp.float32) + bo_ref[...]


def _tco(accp, g, dinv, bg, wo, bo):
    return pl.pallas_call(
        _tco_body,
        grid=(NP // BLK,),
        in_specs=[
            pl.BlockSpec((NW, H, BLK), lambda i: (0, 0, i)),
            pl.BlockSpec((H, BLK), lambda i: (0, i)),
            pl.BlockSpec((1, BLK), lambda i: (0, i)),
            pl.BlockSpec((H, 1), lambda i: (0, 0)),
            pl.BlockSpec((H, C), lambda i: (0, 0)),
            pl.BlockSpec((1, C), lambda i: (0, 0)),
        ],
        out_specs=[
            pl.BlockSpec((BLK, H), lambda i: (i, 0)),
            pl.BlockSpec((BLK, C), lambda i: (i, 0)),
        ],
        out_shape=[
            jax.ShapeDtypeStruct((NP, H), jnp.float32),
            jax.ShapeDtypeStruct((NP, C), jnp.float32),
        ],
    )(accp, g, dinv, bg, wo, bo)


def kernel(x, edge_index, W_gcn, b_gcn, W_out, b_out):
    src = edge_index[0]
    dst = edge_index[1]
    xp = jnp.pad(x, ((0, NP - N), (0, 0)))
    bg = b_gcn.reshape(1, H)
    bo = b_out.reshape(1, C)

    degp = _sc_degree(dst).reshape(NW, NP)
    hp = _tch(xp, W_gcn)
    g, dinv = _tcg(hp, degp)
    accp = _sc_aggregate(src, dst, g.reshape(H * NP))
    hr, z = _tco(accp.reshape(NW, H, NP), g, dinv, bg.T, W_out, bo)

    return (hr[:N], z[:N])


# speedup vs baseline: 1.2540x; 1.2540x over previous
import dataclasses
import functools

import jax
import jax.numpy as jnp
from jax import lax
from jax.experimental import pallas as pl
from jax.experimental.pallas import tpu as pltpu
from jax.experimental.pallas import tpu_sc as plsc

N = 10000
E = 160000
D = 256
H = 3
C = 4

NC, NS = 2, 16
NW = NC * NS
NP = 10240
NSL = NP // NS
EW = E // NW
EWF = (EW // 16) * 16
BLK = 2048
L = 16
EWA = EW + L

_vmesh = plsc.VectorSubcoreMesh(
    core_axis_name="c", subcore_axis_name="s", num_cores=NC, num_subcores=NS
)

_sc_params = pltpu.CompilerParams()
if "needs_layout_passes" in pltpu.CompilerParams.__dataclass_fields__:
    _sc_params = dataclasses.replace(_sc_params, needs_layout_passes=False)


def _tail_mask():
    return lax.iota(jnp.int32, L) < (EW - EWF)


def _slice_sum(stage, acc_sl, tmp, base, count, stride):
    pltpu.sync_copy(stage.at[pl.ds(base, NSL)], acc_sl)

    @pl.loop(1, count)
    def _(k):
        pltpu.sync_copy(stage.at[pl.ds(base + k * stride, NSL)], tmp)

        @pl.loop(0, NSL, step=L)
        def _(i):
            acc_sl[pl.ds(i, L)] = acc_sl[pl.ds(i, L)] + tmp[pl.ds(i, L)]


@functools.partial(
    pl.kernel,
    out_type=jax.ShapeDtypeStruct((NC * NP,), jnp.float32),
    mesh=_vmesh,
    scratch_types=[
        pltpu.VMEM((EWA,), jnp.int32),
        pltpu.VMEM((NP,), jnp.float32),
        pltpu.VMEM((NSL,), jnp.float32),
        pltpu.VMEM((NSL,), jnp.float32),
        pltpu.VMEM_SHARED((NS * NP,), jnp.float32),
        pltpu.SemaphoreType.DMA,
    ],
    compiler_params=_sc_params,
)
def _sc_degree(dst_hbm, out_hbm, dst_v, deg_v, sl_v, tmp_v, stage, sem):
    cid = lax.axis_index("c")
    sid = lax.axis_index("s")
    wid = cid * NS + sid
    cp = pltpu.async_copy(dst_hbm.at[pl.ds(wid * EW, EW)],
                          dst_v.at[pl.ds(0, EW)], sem)

    @pl.loop(0, NP, step=L)
    def _(i):
        deg_v[pl.ds(i, L)] = jnp.zeros((L,), jnp.float32)

    cp.wait()
    ones = jnp.ones((L,), jnp.float32)

    @pl.loop(0, EWF, step=L, unroll=2)
    def _(i):
        plsc.addupdate_scatter(deg_v, [dst_v[pl.ds(i, L)]], ones)

    plsc.addupdate_scatter(
        deg_v, [dst_v[pl.ds(EWF, L)]], ones, mask=_tail_mask()
    )

    pltpu.sync_copy(deg_v, stage.at[pl.ds(sid * NP, NP)])
    plsc.subcore_barrier()
    _slice_sum(stage, sl_v, tmp_v, sid * NSL, NS, NP)
    pltpu.sync_copy(sl_v, out_hbm.at[pl.ds(cid * NP + sid * NSL, NSL)])


@functools.partial(
    pl.kernel,
    out_type=jax.ShapeDtypeStruct((NC * H * NP,), jnp.float32),
    mesh=_vmesh,
    scratch_types=[
        pltpu.VMEM((EWA,), jnp.int32),
        pltpu.VMEM((EWA,), jnp.int32),
        pltpu.VMEM((NP,), jnp.float32),
        pltpu.VMEM((NP,), jnp.float32),
        pltpu.VMEM((NP,), jnp.float32),
        pltpu.VMEM((NP,), jnp.float32),
        pltpu.VMEM((NP,), jnp.float32),
        pltpu.VMEM((NP,), jnp.float32),
        pltpu.VMEM((NSL,), jnp.float32),
        pltpu.VMEM((NSL,), jnp.float32),
        pltpu.VMEM_SHARED((NS * H * NP,), jnp.float32),
        pltpu.SemaphoreType.DMA,
    ],
    compiler_params=_sc_params,
)
def _sc_aggregate(src_hbm, dst_hbm, g_hbm, out_hbm,
                  src_v, dst_v, g0, g1, g2, a0, a1, a2, sl_v, tmp_v,
                  stage, sem):
    cid = lax.axis_index("c")
    sid = lax.axis_index("s")
    wid = cid * NS + sid
    base = wid * EW
    cps = [
        pltpu.async_copy(src_hbm.at[pl.ds(base, EW)],
                         src_v.at[pl.ds(0, EW)], sem),
        pltpu.async_copy(dst_hbm.at[pl.ds(base, EW)],
                         dst_v.at[pl.ds(0, EW)], sem),
        pltpu.async_copy(g_hbm.at[pl.ds(0, NP)], g0, sem),
        pltpu.async_copy(g_hbm.at[pl.ds(NP, NP)], g1, sem),
        pltpu.async_copy(g_hbm.at[pl.ds(2 * NP, NP)], g2, sem),
    ]

    @pl.loop(0, NP, step=L)
    def _(i):
        z = jnp.zeros((L,), jnp.float32)
        a0[pl.ds(i, L)] = z
        a1[pl.ds(i, L)] = z
        a2[pl.ds(i, L)] = z

    for cp in cps:
        cp.wait()

    def edge(i, mask=None):
        s = src_v[pl.ds(i, L)]
        d = dst_v[pl.ds(i, L)]
        plsc.addupdate_scatter(a0, [d], plsc.load_gather(g0, [s], mask=mask),
                               mask=mask)
        plsc.addupdate_scatter(a1, [d], plsc.load_gather(g1, [s], mask=mask),
                               mask=mask)
        plsc.addupdate_scatter(a2, [d], plsc.load_gather(g2, [s], mask=mask),
                               mask=mask)

    @pl.loop(0, EWF, step=L, unroll=2)
    def _(i):
        edge(i)

    edge(EWF, mask=_tail_mask())

    pltpu.sync_copy(a0, stage.at[pl.ds(sid * (H * NP), NP)])
    pltpu.sync_copy(a1, stage.at[pl.ds(sid * (H * NP) + NP, NP)])
    pltpu.sync_copy(a2, stage.at[pl.ds(sid * (H * NP) + 2 * NP, NP)])
    plsc.subcore_barrier()
    for c in range(H):
        _slice_sum(stage, sl_v, tmp_v, c * NP + sid * NSL, NS, H * NP)
        pltpu.sync_copy(
            sl_v,
            out_hbm.at[pl.ds(cid * (H * NP) + c * NP + sid * NSL, NSL)],
        )


def _tc1_body(wt_ref, x_ref, degp_ref, g_ref, dinv_ref):
    deg = degp_ref[0:1, :] + degp_ref[1:2, :] + 1.0
    dinv = lax.rsqrt(deg)
    ht = lax.dot_general(wt_ref[...], x_ref[...],
                         (((1,), (1,)), ((), ())),
                         preferred_element_type=jnp.float32)
    g_ref[...] = ht * dinv
    dinv_ref[...] = dinv


def _tc1(wt, x, degp):
    return pl.pallas_call(
        _tc1_body,
        grid=(NP // BLK,),
        in_specs=[
            pl.BlockSpec((H, D), lambda i: (0, 0)),
            pl.BlockSpec((BLK, D), lambda i: (i, 0)),
            pl.BlockSpec((NC, BLK), lambda i: (0, i)),
        ],
        out_specs=[
            pl.BlockSpec((H, BLK), lambda i: (0, i)),
            pl.BlockSpec((1, BLK), lambda i: (0, i)),
        ],
        out_shape=[
            jax.ShapeDtypeStruct((H, NP), jnp.float32),
            jax.ShapeDtypeStruct((1, NP), jnp.float32),
        ],
    )(wt, x, degp)


def _tc2_body(accp_ref, g_ref, dinv_ref, bg_ref, wo_ref, bo_ref,
              hr_ref, z_ref):
    a = accp_ref[...]
    acc = a[0:H, :] + a[H:2 * H, :] + g_ref[...]
    hg = acc * dinv_ref[...] + bg_ref[...]
    hr = jnp.maximum(hg, 0.0)
    hr_row = hr.T
    hr_ref[...] = hr_row
    z_ref[...] = jnp.dot(hr_row, wo_ref[...],
                         preferred_element_type=jnp.float32) + bo_ref[...]


def _tc2(accp, g, dinv, bg, wo, bo):
    return pl.pallas_call(
        _tc2_body,
        grid=(NP // BLK,),
        in_specs=[
            pl.BlockSpec((NC * H, BLK), lambda i: (0, i)),
            pl.BlockSpec((H, BLK), lambda i: (0, i)),
            pl.BlockSpec((1, BLK), lambda i: (0, i)),
            pl.BlockSpec((H, 1), lambda i: (0, 0)),
            pl.BlockSpec((H, C), lambda i: (0, 0)),
            pl.BlockSpec((1, C), lambda i: (0, 0)),
        ],
        out_specs=[
            pl.BlockSpec((BLK, H), lambda i: (i, 0)),
            pl.BlockSpec((BLK, C), lambda i: (i, 0)),
        ],
        out_shape=[
            jax.ShapeDtypeStruct((N, H), jnp.float32),
            jax.ShapeDtypeStruct((N, C), jnp.float32),
        ],
    )(accp, g, dinv, bg, wo, bo)


def kernel(x, edge_index, W_gcn, b_gcn, W_out, b_out):
    src = edge_index[0]
    dst = edge_index[1]
    wt = W_gcn.T
    bg = b_gcn.reshape(H, 1)
    bo = b_out.reshape(1, C)

    degp = _sc_degree(dst).reshape(NC, NP)
    g, dinv = _tc1(wt, x, degp)
    accp = _sc_aggregate(src, dst, g.reshape(H * NP))
    hr, z = _tc2(accp.reshape(NC * H, NP), g, dinv, bg, W_out, bo)
    return (hr, z)


# speedup vs baseline: 1.4453x; 1.1525x over previous
import dataclasses
import functools

import jax
import jax.numpy as jnp
from jax import lax
from jax.experimental import pallas as pl
from jax.experimental.pallas import tpu as pltpu
from jax.experimental.pallas import tpu_sc as plsc

N = 10000
E = 160000
D = 256
H = 3
C = 4

NC, NS = 2, 16
NW = NC * NS
NP = 10240
NSL = NP // NS
EW = E // NW
EWF = (EW // 16) * 16
BLK = 2048
L = 16
EWA = EW + L

_vmesh = plsc.VectorSubcoreMesh(
    core_axis_name="c", subcore_axis_name="s", num_cores=NC, num_subcores=NS
)

_sc_params = pltpu.CompilerParams()
if "needs_layout_passes" in pltpu.CompilerParams.__dataclass_fields__:
    _sc_params = dataclasses.replace(_sc_params, needs_layout_passes=False)


def _tail_mask():
    return lax.iota(jnp.int32, L) < (EW - EWF)


def _fetch_partials(stage, buf, sem, base, count, stride):
    cps = [
        pltpu.async_copy(stage.at[pl.ds(base + k * stride, NSL)],
                         buf.at[pl.ds(k * NSL, NSL)], sem)
        for k in range(count)
    ]
    return cps


def _reduce_partials(buf, out_sl, count, off):
    @pl.loop(0, NSL, step=L)
    def _(i):
        v = buf[pl.ds(off + i, L)]
        for k in range(1, count):
            v = v + buf[pl.ds(off + k * NSL + i, L)]
        out_sl[pl.ds(i, L)] = v


@functools.partial(
    pl.kernel,
    out_type=jax.ShapeDtypeStruct((NC * NP,), jnp.float32),
    mesh=_vmesh,
    scratch_types=[
        pltpu.VMEM((EWA,), jnp.int32),
        pltpu.VMEM((NP,), jnp.float32),
        pltpu.VMEM((NSL,), jnp.float32),
        pltpu.VMEM((NS * NSL,), jnp.float32),
        pltpu.VMEM_SHARED((NS * NP,), jnp.float32),
        pltpu.SemaphoreType.DMA,
    ],
    compiler_params=_sc_params,
)
def _sc_degree(dst_hbm, out_hbm, dst_v, deg_v, sl_v, buf_v, stage, sem):
    cid = lax.axis_index("c")
    sid = lax.axis_index("s")
    wid = cid * NS + sid
    cp = pltpu.async_copy(dst_hbm.at[pl.ds(wid * EW, EW)],
                          dst_v.at[pl.ds(0, EW)], sem)

    @pl.loop(0, NP, step=L)
    def _(i):
        deg_v[pl.ds(i, L)] = jnp.zeros((L,), jnp.float32)

    cp.wait()
    ones = jnp.ones((L,), jnp.float32)

    @pl.loop(0, EWF, step=L, unroll=4)
    def _(i):
        plsc.addupdate_scatter(deg_v, [dst_v[pl.ds(i, L)]], ones)

    plsc.addupdate_scatter(
        deg_v, [dst_v[pl.ds(EWF, L)]], ones, mask=_tail_mask()
    )

    pltpu.sync_copy(deg_v, stage.at[pl.ds(sid * NP, NP)])
    plsc.subcore_barrier()
    for cp in _fetch_partials(stage, buf_v, sem, sid * NSL, NS, NP):
        cp.wait()
    _reduce_partials(buf_v, sl_v, NS, 0)
    pltpu.sync_copy(sl_v, out_hbm.at[pl.ds(cid * NP + sid * NSL, NSL)])


@functools.partial(
    pl.kernel,
    out_type=jax.ShapeDtypeStruct((NW * H * NP,), jnp.float32),
    mesh=_vmesh,
    scratch_types=[
        pltpu.VMEM((EWA,), jnp.int32),
        pltpu.VMEM((EWA,), jnp.int32),
        pltpu.VMEM((NP,), jnp.float32),
        pltpu.VMEM((NP,), jnp.float32),
        pltpu.VMEM((NP,), jnp.float32),
        pltpu.VMEM((NP,), jnp.float32),
        pltpu.VMEM((NP,), jnp.float32),
        pltpu.VMEM((NP,), jnp.float32),
        pltpu.SemaphoreType.DMA,
    ],
    compiler_params=_sc_params,
)
def _sc_aggregate(src_hbm, dst_hbm, g_hbm, out_hbm,
                  src_v, dst_v, g0, g1, g2, a0, a1, a2, sem):
    cid = lax.axis_index("c")
    sid = lax.axis_index("s")
    wid = cid * NS + sid
    base = wid * EW
    cps = [
        pltpu.async_copy(src_hbm.at[pl.ds(base, EW)],
                         src_v.at[pl.ds(0, EW)], sem),
        pltpu.async_copy(dst_hbm.at[pl.ds(base, EW)],
                         dst_v.at[pl.ds(0, EW)], sem),
        pltpu.async_copy(g_hbm.at[pl.ds(0, NP)], g0, sem),
        pltpu.async_copy(g_hbm.at[pl.ds(NP, NP)], g1, sem),
        pltpu.async_copy(g_hbm.at[pl.ds(2 * NP, NP)], g2, sem),
    ]

    @pl.loop(0, NP, step=L)
    def _(i):
        z = jnp.zeros((L,), jnp.float32)
        a0[pl.ds(i, L)] = z
        a1[pl.ds(i, L)] = z
        a2[pl.ds(i, L)] = z

    for cp in cps:
        cp.wait()

    def edge(i, mask=None):
        s = src_v[pl.ds(i, L)]
        d = dst_v[pl.ds(i, L)]
        plsc.addupdate_scatter(a0, [d], plsc.load_gather(g0, [s], mask=mask),
                               mask=mask)
        plsc.addupdate_scatter(a1, [d], plsc.load_gather(g1, [s], mask=mask),
                               mask=mask)
        plsc.addupdate_scatter(a2, [d], plsc.load_gather(g2, [s], mask=mask),
                               mask=mask)

    @pl.loop(0, EWF, step=L, unroll=4)
    def _(i):
        edge(i)

    edge(EWF, mask=_tail_mask())

    obase = wid * (H * NP)
    pltpu.sync_copy(a0, out_hbm.at[pl.ds(obase, NP)])
    pltpu.sync_copy(a1, out_hbm.at[pl.ds(obase + NP, NP)])
    pltpu.sync_copy(a2, out_hbm.at[pl.ds(obase + 2 * NP, NP)])


def _tc1_body(wt_ref, x_ref, degp_ref, g_ref, dinv_ref):
    deg = degp_ref[0:1, :] + degp_ref[1:2, :] + 1.0
    dinv = lax.rsqrt(deg)
    ht = lax.dot_general(wt_ref[...], x_ref[...],
                         (((1,), (1,)), ((), ())),
                         preferred_element_type=jnp.float32)
    g_ref[...] = ht * dinv
    dinv_ref[...] = dinv


def _tc1(wt, x, degp):
    return pl.pallas_call(
        _tc1_body,
        grid=(NP // BLK,),
        in_specs=[
            pl.BlockSpec((H, D), lambda i: (0, 0)),
            pl.BlockSpec((BLK, D), lambda i: (i, 0)),
            pl.BlockSpec((NC, BLK), lambda i: (0, i)),
        ],
        out_specs=[
            pl.BlockSpec((H, BLK), lambda i: (0, i)),
            pl.BlockSpec((1, BLK), lambda i: (0, i)),
        ],
        out_shape=[
            jax.ShapeDtypeStruct((H, NP), jnp.float32),
            jax.ShapeDtypeStruct((1, NP), jnp.float32),
        ],
    )(wt, x, degp)


def _tc2_body(accp_ref, g_ref, dinv_ref, bg_ref, wo_ref, bo_ref,
              hr_ref, z_ref):
    a = accp_ref[...]
    acc = a.reshape(NW, H, a.shape[-1]).sum(axis=0) + g_ref[...]
    hg = acc * dinv_ref[...] + bg_ref[...]
    hr = jnp.maximum(hg, 0.0)
    hr_row = hr.T
    hr_ref[...] = hr_row
    z_ref[...] = jnp.dot(hr_row, wo_ref[...],
                         preferred_element_type=jnp.float32) + bo_ref[...]


def _tc2(accp, g, dinv, bg, wo, bo):
    return pl.pallas_call(
        _tc2_body,
        grid=(NP // BLK,),
        in_specs=[
            pl.BlockSpec((NW * H, BLK), lambda i: (0, i)),
            pl.BlockSpec((H, BLK), lambda i: (0, i)),
            pl.BlockSpec((1, BLK), lambda i: (0, i)),
            pl.BlockSpec((H, 1), lambda i: (0, 0)),
            pl.BlockSpec((H, C), lambda i: (0, 0)),
            pl.BlockSpec((1, C), lambda i: (0, 0)),
        ],
        out_specs=[
            pl.BlockSpec((BLK, H), lambda i: (i, 0)),
            pl.BlockSpec((BLK, C), lambda i: (i, 0)),
        ],
        out_shape=[
            jax.ShapeDtypeStruct((N, H), jnp.float32),
            jax.ShapeDtypeStruct((N, C), jnp.float32),
        ],
    )(accp, g, dinv, bg, wo, bo)


def kernel(x, edge_index, W_gcn, b_gcn, W_out, b_out):
    src = edge_index[0]
    dst = edge_index[1]
    wt = W_gcn.T
    bg = b_gcn.reshape(H, 1)
    bo = b_out.reshape(1, C)

    degp = _sc_degree(dst).reshape(NC, NP)
    g, dinv = _tc1(wt, x, degp)
    accp = _sc_aggregate(src, dst, g.reshape(H * NP))
    hr, z = _tc2(accp.reshape(NW * H, NP), g, dinv, bg, W_out, bo)
    return (hr, z)


# speedup vs baseline: 1.4623x; 1.0117x over previous
import dataclasses
import functools

import jax
import jax.numpy as jnp
from jax import lax
from jax.experimental import pallas as pl
from jax.experimental.pallas import tpu as pltpu
from jax.experimental.pallas import tpu_sc as plsc

N = 10000
E = 160000
D = 256
H = 3
C = 4

NC, NS = 2, 16
NW = NC * NS
NP = 10240
NSL = NP // NS
EW = E // NW
EWF = (EW // 16) * 16
BLK = 2048
L = 16
EWA = EW + L

_vmesh = plsc.VectorSubcoreMesh(
    core_axis_name="c", subcore_axis_name="s", num_cores=NC, num_subcores=NS
)

_sc_params = pltpu.CompilerParams()
if "needs_layout_passes" in pltpu.CompilerParams.__dataclass_fields__:
    _sc_params = dataclasses.replace(_sc_params, needs_layout_passes=False)


def _tail_mask():
    return lax.iota(jnp.int32, L) < (EW - EWF)


def _fetch_partials(stage, buf, sem, base, count, stride):
    cps = [
        pltpu.async_copy(stage.at[pl.ds(base + k * stride, NSL)],
                         buf.at[pl.ds(k * NSL, NSL)], sem)
        for k in range(count)
    ]
    return cps


def _reduce_partials(buf, out_sl, count, off):
    @pl.loop(0, NSL, step=L)
    def _(i):
        v = buf[pl.ds(off + i, L)]
        for k in range(1, count):
            v = v + buf[pl.ds(off + k * NSL + i, L)]
        out_sl[pl.ds(i, L)] = v


@functools.partial(
    pl.kernel,
    out_type=jax.ShapeDtypeStruct((NC * NP,), jnp.float32),
    mesh=_vmesh,
    scratch_types=[
        pltpu.VMEM((EWA,), jnp.int32),
        pltpu.VMEM((NP,), jnp.float32),
        pltpu.VMEM((NSL,), jnp.float32),
        pltpu.VMEM((NS * NSL,), jnp.float32),
        pltpu.VMEM_SHARED((NS * NP,), jnp.float32),
        pltpu.SemaphoreType.DMA,
    ],
    compiler_params=_sc_params,
)
def _sc_degree(dst_hbm, out_hbm, dst_v, deg_v, sl_v, buf_v, stage, sem):
    cid = lax.axis_index("c")
    sid = lax.axis_index("s")
    wid = cid * NS + sid
    cp = pltpu.async_copy(dst_hbm.at[pl.ds(wid * EW, EW)],
                          dst_v.at[pl.ds(0, EW)], sem)

    @pl.loop(0, NP, step=L)
    def _(i):
        deg_v[pl.ds(i, L)] = jnp.zeros((L,), jnp.float32)

    cp.wait()
    ones = jnp.ones((L,), jnp.float32)

    @pl.loop(0, EWF, step=L, unroll=4)
    def _(i):
        plsc.addupdate_scatter(deg_v, [dst_v[pl.ds(i, L)]], ones)

    plsc.addupdate_scatter(
        deg_v, [dst_v[pl.ds(EWF, L)]], ones, mask=_tail_mask()
    )

    pltpu.sync_copy(deg_v, stage.at[pl.ds(sid * NP, NP)])
    plsc.subcore_barrier()
    for cp in _fetch_partials(stage, buf_v, sem, sid * NSL, NS, NP):
        cp.wait()
    _reduce_partials(buf_v, sl_v, NS, 0)
    pltpu.sync_copy(sl_v, out_hbm.at[pl.ds(cid * NP + sid * NSL, NSL)])


@functools.partial(
    pl.kernel,
    out_type=jax.ShapeDtypeStruct((NC * H * NP,), jnp.float32),
    mesh=_vmesh,
    scratch_types=[
        pltpu.VMEM((EWA,), jnp.int32),
        pltpu.VMEM((EWA,), jnp.int32),
        pltpu.VMEM((NP,), jnp.float32),
        pltpu.VMEM((NP,), jnp.float32),
        pltpu.VMEM((NP,), jnp.float32),
        pltpu.VMEM((NP,), jnp.float32),
        pltpu.VMEM((NP,), jnp.float32),
        pltpu.VMEM((NP,), jnp.float32),
        pltpu.VMEM((NSL,), jnp.float32),
        pltpu.VMEM((NS * NSL,), jnp.float32),
        pltpu.VMEM_SHARED((NS * NP,), jnp.float32),
        pltpu.SemaphoreType.DMA,
    ],
    compiler_params=_sc_params,
)
def _sc_aggregate(src_hbm, dst_hbm, g_hbm, out_hbm,
                  src_v, dst_v, g0, g1, g2, a0, a1, a2, sl_v, buf_v,
                  stage, sem):
    cid = lax.axis_index("c")
    sid = lax.axis_index("s")
    wid = cid * NS + sid
    base = wid * EW
    cps = [
        pltpu.async_copy(src_hbm.at[pl.ds(base, EW)],
                         src_v.at[pl.ds(0, EW)], sem),
        pltpu.async_copy(dst_hbm.at[pl.ds(base, EW)],
                         dst_v.at[pl.ds(0, EW)], sem),
        pltpu.async_copy(g_hbm.at[pl.ds(0, NP)], g0, sem),
        pltpu.async_copy(g_hbm.at[pl.ds(NP, NP)], g1, sem),
        pltpu.async_copy(g_hbm.at[pl.ds(2 * NP, NP)], g2, sem),
    ]

    @pl.loop(0, NP, step=L)
    def _(i):
        z = jnp.zeros((L,), jnp.float32)
        a0[pl.ds(i, L)] = z
        a1[pl.ds(i, L)] = z
        a2[pl.ds(i, L)] = z

    for cp in cps:
        cp.wait()

    def edge(i, mask=None):
        s = src_v[pl.ds(i, L)]
        d = dst_v[pl.ds(i, L)]
        plsc.addupdate_scatter(a0, [d], plsc.load_gather(g0, [s], mask=mask),
                               mask=mask)
        plsc.addupdate_scatter(a1, [d], plsc.load_gather(g1, [s], mask=mask),
                               mask=mask)
        plsc.addupdate_scatter(a2, [d], plsc.load_gather(g2, [s], mask=mask),
                               mask=mask)

    @pl.loop(0, EWF, step=L, unroll=4)
    def _(i):
        edge(i)

    edge(EWF, mask=_tail_mask())

    for c, a in ((0, a0), (1, a1), (2, a2)):
        pltpu.sync_copy(a, stage.at[pl.ds(sid * NP, NP)])
        plsc.subcore_barrier()
        for cp in _fetch_partials(stage, buf_v, sem, sid * NSL, NS, NP):
            cp.wait()
        _reduce_partials(buf_v, sl_v, NS, 0)
        pltpu.sync_copy(
            sl_v,
            out_hbm.at[pl.ds(cid * (H * NP) + c * NP + sid * NSL, NSL)],
        )
        plsc.subcore_barrier()


def _tc1_body(wt_ref, x_ref, degp_ref, g_ref, dinv_ref):
    deg = degp_ref[0:1, :] + degp_ref[1:2, :] + 1.0
    dinv = lax.rsqrt(deg)
    ht = lax.dot_general(wt_ref[...], x_ref[...],
                         (((1,), (1,)), ((), ())),
                         preferred_element_type=jnp.float32)
    g_ref[...] = ht * dinv
    dinv_ref[...] = dinv


def _tc1(wt, x, degp):
    return pl.pallas_call(
        _tc1_body,
        grid=(NP // BLK,),
        in_specs=[
            pl.BlockSpec((H, D), lambda i: (0, 0)),
            pl.BlockSpec((BLK, D), lambda i: (i, 0)),
            pl.BlockSpec((NC, BLK), lambda i: (0, i)),
        ],
        out_specs=[
            pl.BlockSpec((H, BLK), lambda i: (0, i)),
            pl.BlockSpec((1, BLK), lambda i: (0, i)),
        ],
        out_shape=[
            jax.ShapeDtypeStruct((H, NP), jnp.float32),
            jax.ShapeDtypeStruct((1, NP), jnp.float32),
        ],
    )(wt, x, degp)


def _tc2_body(accp_ref, g_ref, dinv_ref, bg_ref, wo_ref, bo_ref,
              hr_ref, z_ref):
    a = accp_ref[...]
    acc = a[0:H, :] + a[H:2 * H, :] + g_ref[...]
    hg = acc * dinv_ref[...] + bg_ref[...]
    hr = jnp.maximum(hg, 0.0)
    hr_row = hr.T
    hr_ref[...] = hr_row
    z_ref[...] = jnp.dot(hr_row, wo_ref[...],
                         preferred_element_type=jnp.float32) + bo_ref[...]


def _tc2(accp, g, dinv, bg, wo, bo):
    return pl.pallas_call(
        _tc2_body,
        grid=(NP // BLK,),
        in_specs=[
            pl.BlockSpec((NC * H, BLK), lambda i: (0, i)),
            pl.BlockSpec((H, BLK), lambda i: (0, i)),
            pl.BlockSpec((1, BLK), lambda i: (0, i)),
            pl.BlockSpec((H, 1), lambda i: (0, 0)),
            pl.BlockSpec((H, C), lambda i: (0, 0)),
            pl.BlockSpec((1, C), lambda i: (0, 0)),
        ],
        out_specs=[
            pl.BlockSpec((BLK, H), lambda i: (i, 0)),
            pl.BlockSpec((BLK, C), lambda i: (i, 0)),
        ],
        out_shape=[
            jax.ShapeDtypeStruct((N, H), jnp.float32),
            jax.ShapeDtypeStruct((N, C), jnp.float32),
        ],
    )(accp, g, dinv, bg, wo, bo)


def kernel(x, edge_index, W_gcn, b_gcn, W_out, b_out):
    src = edge_index[0]
    dst = edge_index[1]
    wt = W_gcn.T
    bg = b_gcn.reshape(H, 1)
    bo = b_out.reshape(1, C)

    degp = _sc_degree(dst).reshape(NC, NP)
    g, dinv = _tc1(wt, x, degp)
    accp = _sc_aggregate(src, dst, g.reshape(H * NP))
    hr, z = _tc2(accp.reshape(NC * H, NP), g, dinv, bg, W_out, bo)
    return (hr, z)


# speedup vs baseline: 1.5610x; 1.0675x over previous
import dataclasses
import functools

import jax
import jax.numpy as jnp
from jax import lax
from jax.experimental import pallas as pl
from jax.experimental.pallas import tpu as pltpu
from jax.experimental.pallas import tpu_sc as plsc

N = 10000
E = 160000
D = 256
H = 3
C = 4

NC, NS = 2, 16
NW = NC * NS
NP = 10240
NSL = NP // NS
EW = E // NW
EWF = (EW // 16) * 16
BLK = 2048
L = 16
EWW = 5120

_vmesh = plsc.VectorSubcoreMesh(
    core_axis_name="c", subcore_axis_name="s", num_cores=NC, num_subcores=NS
)

_sc_params = pltpu.CompilerParams()
if "needs_layout_passes" in pltpu.CompilerParams.__dataclass_fields__:
    _sc_params = dataclasses.replace(_sc_params, needs_layout_passes=False)


def _edge_window(wid):
    base = wid * EW
    off = lax.rem(base, 128)
    c0 = pl.multiple_of(base - off, 128)
    o16 = off - lax.rem(off, L)
    fs = off - o16
    iota = lax.iota(jnp.int32, L)
    m_first = iota >= fs
    m_last = iota < fs + (EW - EWF)
    return c0, o16, m_first, m_last


def _fetch_partials(stage, buf, sem, base, count, stride):
    cps = [
        pltpu.async_copy(stage.at[pl.ds(base + k * stride, NSL)],
                         buf.at[pl.ds(k * NSL, NSL)], sem)
        for k in range(count)
    ]
    return cps


def _reduce_partials(buf, out_sl, count, off):
    @pl.loop(0, NSL, step=L)
    def _(i):
        v = buf[pl.ds(off + i, L)]
        for k in range(1, count):
            v = v + buf[pl.ds(off + k * NSL + i, L)]
        out_sl[pl.ds(i, L)] = v


@functools.partial(
    pl.kernel,
    out_type=jax.ShapeDtypeStruct((NC * NP,), jnp.float32),
    mesh=_vmesh,
    scratch_types=[
        pltpu.VMEM((2, EWW), jnp.int32),
        pltpu.VMEM((NP,), jnp.float32),
        pltpu.VMEM((NSL,), jnp.float32),
        pltpu.VMEM((NS * NSL,), jnp.float32),
        pltpu.VMEM_SHARED((NS * NP,), jnp.float32),
        pltpu.SemaphoreType.DMA,
    ],
    compiler_params=_sc_params,
)
def _sc_degree(ei_hbm, out_hbm, ei_v, deg_v, sl_v, buf_v, stage, sem):
    cid = lax.axis_index("c")
    sid = lax.axis_index("s")
    wid = cid * NS + sid
    c0, o16, m_first, m_last = _edge_window(wid)
    cp = pltpu.async_copy(ei_hbm.at[:, pl.ds(c0, EWW)],
                          ei_v.at[:, pl.ds(0, EWW)], sem)

    @pl.loop(0, NP, step=L)
    def _(i):
        deg_v[pl.ds(i, L)] = jnp.zeros((L,), jnp.float32)

    cp.wait()
    ones = jnp.ones((L,), jnp.float32)

    def count(i, mask=None):
        plsc.addupdate_scatter(deg_v, [ei_v[1, pl.ds(o16 + i, L)]], ones,
                               mask=mask)

    count(0, mask=m_first)

    @pl.loop(L, EWF, step=L, unroll=4)
    def _(i):
        count(i)

    count(EWF, mask=m_last)

    pltpu.sync_copy(deg_v, stage.at[pl.ds(sid * NP, NP)])
    plsc.subcore_barrier()
    for cp in _fetch_partials(stage, buf_v, sem, sid * NSL, NS, NP):
        cp.wait()
    _reduce_partials(buf_v, sl_v, NS, 0)
    pltpu.sync_copy(sl_v, out_hbm.at[pl.ds(cid * NP + sid * NSL, NSL)])


@functools.partial(
    pl.kernel,
    out_type=jax.ShapeDtypeStruct((NC * H * NP,), jnp.float32),
    mesh=_vmesh,
    scratch_types=[
        pltpu.VMEM((2, EWW), jnp.int32),
        pltpu.VMEM((NP,), jnp.float32),
        pltpu.VMEM((NP,), jnp.float32),
        pltpu.VMEM((NP,), jnp.float32),
        pltpu.VMEM((NP,), jnp.float32),
        pltpu.VMEM((NP,), jnp.float32),
        pltpu.VMEM((NP,), jnp.float32),
        pltpu.VMEM((NSL,), jnp.float32),
        pltpu.VMEM((NS * NSL,), jnp.float32),
        pltpu.VMEM_SHARED((NS * NP,), jnp.float32),
        pltpu.SemaphoreType.DMA,
    ],
    compiler_params=_sc_params,
)
def _sc_aggregate(ei_hbm, g_hbm, out_hbm,
                  ei_v, g0, g1, g2, a0, a1, a2, sl_v, buf_v,
                  stage, sem):
    cid = lax.axis_index("c")
    sid = lax.axis_index("s")
    wid = cid * NS + sid
    c0, o16, m_first, m_last = _edge_window(wid)
    cps = [
        pltpu.async_copy(ei_hbm.at[:, pl.ds(c0, EWW)],
                         ei_v.at[:, pl.ds(0, EWW)], sem),
        pltpu.async_copy(g_hbm.at[pl.ds(0, NP)], g0, sem),
        pltpu.async_copy(g_hbm.at[pl.ds(NP, NP)], g1, sem),
        pltpu.async_copy(g_hbm.at[pl.ds(2 * NP, NP)], g2, sem),
    ]

    @pl.loop(0, NP, step=L)
    def _(i):
        z = jnp.zeros((L,), jnp.float32)
        a0[pl.ds(i, L)] = z
        a1[pl.ds(i, L)] = z
        a2[pl.ds(i, L)] = z

    for cp in cps:
        cp.wait()

    def edge(i, mask=None):
        s = ei_v[0, pl.ds(o16 + i, L)]
        d = ei_v[1, pl.ds(o16 + i, L)]
        plsc.addupdate_scatter(a0, [d], plsc.load_gather(g0, [s], mask=mask),
                               mask=mask)
        plsc.addupdate_scatter(a1, [d], plsc.load_gather(g1, [s], mask=mask),
                               mask=mask)
        plsc.addupdate_scatter(a2, [d], plsc.load_gather(g2, [s], mask=mask),
                               mask=mask)

    edge(0, mask=m_first)

    @pl.loop(L, EWF, step=L, unroll=4)
    def _(i):
        edge(i)

    edge(EWF, mask=m_last)

    for c, a in ((0, a0), (1, a1), (2, a2)):
        pltpu.sync_copy(a, stage.at[pl.ds(sid * NP, NP)])
        plsc.subcore_barrier()
        for cp in _fetch_partials(stage, buf_v, sem, sid * NSL, NS, NP):
            cp.wait()
        _reduce_partials(buf_v, sl_v, NS, 0)
        pltpu.sync_copy(
            sl_v,
            out_hbm.at[pl.ds(cid * (H * NP) + c * NP + sid * NSL, NSL)],
        )
        plsc.subcore_barrier()


def _tc1_body(wt_ref, x_ref, degp_ref, g_ref, dinv_ref):
    deg = degp_ref[0:1, :] + degp_ref[1:2, :] + 1.0
    dinv = lax.rsqrt(deg)
    ht = lax.dot_general(wt_ref[...], x_ref[...],
                         (((1,), (1,)), ((), ())),
                         preferred_element_type=jnp.float32)
    g_ref[...] = ht * dinv
    dinv_ref[...] = dinv


def _tc1(wt, x, degp):
    return pl.pallas_call(
        _tc1_body,
        grid=(NP // BLK,),
        in_specs=[
            pl.BlockSpec((H, D), lambda i: (0, 0)),
            pl.BlockSpec((BLK, D), lambda i: (i, 0)),
            pl.BlockSpec((NC, BLK), lambda i: (0, i)),
        ],
        out_specs=[
            pl.BlockSpec((H, BLK), lambda i: (0, i)),
            pl.BlockSpec((1, BLK), lambda i: (0, i)),
        ],
        out_shape=[
            jax.ShapeDtypeStruct((H, NP), jnp.float32),
            jax.ShapeDtypeStruct((1, NP), jnp.float32),
        ],
    )(wt, x, degp)


def _tc2_body(accp_ref, g_ref, dinv_ref, bg_ref, wo_ref, bo_ref,
              hr_ref, z_ref):
    a = accp_ref[...]
    acc = a[0:H, :] + a[H:2 * H, :] + g_ref[...]
    hg = acc * dinv_ref[...] + bg_ref[...]
    hr = jnp.maximum(hg, 0.0)
    hr_row = hr.T
    hr_ref[...] = hr_row
    z_ref[...] = jnp.dot(hr_row, wo_ref[...],
                         preferred_element_type=jnp.float32) + bo_ref[...]


def _tc2(accp, g, dinv, bg, wo, bo):
    return pl.pallas_call(
        _tc2_body,
        grid=(NP // BLK,),
        in_specs=[
            pl.BlockSpec((NC * H, BLK), lambda i: (0, i)),
            pl.BlockSpec((H, BLK), lambda i: (0, i)),
            pl.BlockSpec((1, BLK), lambda i: (0, i)),
            pl.BlockSpec((H, 1), lambda i: (0, 0)),
            pl.BlockSpec((H, C), lambda i: (0, 0)),
            pl.BlockSpec((1, C), lambda i: (0, 0)),
        ],
        out_specs=[
            pl.BlockSpec((BLK, H), lambda i: (i, 0)),
            pl.BlockSpec((BLK, C), lambda i: (i, 0)),
        ],
        out_shape=[
            jax.ShapeDtypeStruct((N, H), jnp.float32),
            jax.ShapeDtypeStruct((N, C), jnp.float32),
        ],
    )(accp, g, dinv, bg, wo, bo)


def kernel(x, edge_index, W_gcn, b_gcn, W_out, b_out):
    wt = W_gcn.T
    bg = b_gcn.reshape(H, 1)
    bo = b_out.reshape(1, C)

    degp = _sc_degree(edge_index).reshape(NC, NP)
    g, dinv = _tc1(wt, x, degp)
    accp = _sc_aggregate(edge_index, g.reshape(H * NP))
    hr, z = _tc2(accp.reshape(NC * H, NP), g, dinv, bg, W_out, bo)
    return (hr, z)


# speedup vs baseline: 1.6263x; 1.0418x over previous
import dataclasses
import functools

import jax
import jax.numpy as jnp
from jax import lax
from jax.experimental import pallas as pl
from jax.experimental.pallas import tpu as pltpu
from jax.experimental.pallas import tpu_sc as plsc

N = 10000
E = 160000
D = 256
H = 3
C = 4

NC, NS = 2, 16
NW = NC * NS
NP = 10240
NSL = NP // NS
EW = E // NW
EWF = (EW // 16) * 16
BLK = 2048
L = 16
EWW = 5120

_vmesh = plsc.VectorSubcoreMesh(
    core_axis_name="c", subcore_axis_name="s", num_cores=NC, num_subcores=NS
)

_sc_params = pltpu.CompilerParams()
if "needs_layout_passes" in pltpu.CompilerParams.__dataclass_fields__:
    _sc_params = dataclasses.replace(_sc_params, needs_layout_passes=False)


def _edge_window(wid):
    base = wid * EW
    off = lax.rem(base, 128)
    c0 = pl.multiple_of(base - off, 128)
    o16 = off - lax.rem(off, L)
    fs = off - o16
    iota = lax.iota(jnp.int32, L)
    m_first = iota >= fs
    m_last = iota < fs + (EW - EWF)
    return c0, o16, m_first, m_last


def _fetch_partials(stage, buf, sem, base, count, stride):
    cps = [
        pltpu.async_copy(stage.at[pl.ds(base + k * stride, NSL)],
                         buf.at[pl.ds(k * NSL, NSL)], sem)
        for k in range(count)
    ]
    return cps


def _reduce_partials(buf, out_sl, count, off):
    @pl.loop(0, NSL, step=L)
    def _(i):
        v = buf[pl.ds(off + i, L)]
        for k in range(1, count):
            v = v + buf[pl.ds(off + k * NSL + i, L)]
        out_sl[pl.ds(i, L)] = v


@functools.partial(
    pl.kernel,
    out_type=jax.ShapeDtypeStruct((NC * NP,), jnp.float32),
    mesh=_vmesh,
    scratch_types=[
        pltpu.VMEM((2, EWW), jnp.int32),
        pltpu.VMEM((NP,), jnp.float32),
        pltpu.VMEM((NSL,), jnp.float32),
        pltpu.VMEM((NS * NSL,), jnp.float32),
        pltpu.VMEM_SHARED((NS * NP,), jnp.float32),
        pltpu.SemaphoreType.DMA,
    ],
    compiler_params=_sc_params,
)
def _sc_degree(ei_hbm, out_hbm, ei_v, deg_v, sl_v, buf_v, stage, sem):
    cid = lax.axis_index("c")
    sid = lax.axis_index("s")
    wid = cid * NS + sid
    c0, o16, m_first, m_last = _edge_window(wid)
    cp = pltpu.async_copy(ei_hbm.at[:, pl.ds(c0, EWW)],
                          ei_v.at[:, pl.ds(0, EWW)], sem)

    @pl.loop(0, NP, step=L)
    def _(i):
        deg_v[pl.ds(i, L)] = jnp.zeros((L,), jnp.float32)

    cp.wait()
    ones = jnp.ones((L,), jnp.float32)

    def count(i, mask=None):
        plsc.addupdate_scatter(deg_v, [ei_v[1, pl.ds(o16 + i, L)]], ones,
                               mask=mask)

    count(0, mask=m_first)

    @pl.loop(L, EWF, step=L, unroll=4)
    def _(i):
        count(i)

    count(EWF, mask=m_last)

    pltpu.sync_copy(deg_v, stage.at[pl.ds(sid * NP, NP)])
    plsc.subcore_barrier()
    for cp in _fetch_partials(stage, buf_v, sem, sid * NSL, NS, NP):
        cp.wait()
    _reduce_partials(buf_v, sl_v, NS, 0)
    pltpu.sync_copy(sl_v, out_hbm.at[pl.ds(cid * NP + sid * NSL, NSL)])


@functools.partial(
    pl.kernel,
    out_type=jax.ShapeDtypeStruct((NC * H * NP,), jnp.float32),
    mesh=_vmesh,
    scratch_types=[
        pltpu.VMEM((2, EWW), jnp.int32),
        pltpu.VMEM((NP,), jnp.float32),
        pltpu.VMEM((NP,), jnp.float32),
        pltpu.VMEM((NP,), jnp.float32),
        pltpu.VMEM((NP,), jnp.float32),
        pltpu.VMEM((NP,), jnp.float32),
        pltpu.VMEM((NP,), jnp.float32),
        pltpu.VMEM((NSL,), jnp.float32),
        pltpu.VMEM((NS * NSL,), jnp.float32),
        pltpu.VMEM_SHARED((NS * NP,), jnp.float32),
        pltpu.SemaphoreType.DMA,
    ],
    compiler_params=_sc_params,
)
def _sc_aggregate(ei_hbm, g_hbm, out_hbm,
                  ei_v, g0, g1, g2, a0, a1, a2, sl_v, buf_v,
                  stage, sem):
    cid = lax.axis_index("c")
    sid = lax.axis_index("s")
    wid = cid * NS + sid
    c0, o16, m_first, m_last = _edge_window(wid)
    cps = [
        pltpu.async_copy(ei_hbm.at[:, pl.ds(c0, EWW)],
                         ei_v.at[:, pl.ds(0, EWW)], sem),
        pltpu.async_copy(g_hbm.at[pl.ds(0, NP)], g0, sem),
        pltpu.async_copy(g_hbm.at[pl.ds(NP, NP)], g1, sem),
        pltpu.async_copy(g_hbm.at[pl.ds(2 * NP, NP)], g2, sem),
    ]

    @pl.loop(0, NP, step=L)
    def _(i):
        z = jnp.zeros((L,), jnp.float32)
        a0[pl.ds(i, L)] = z
        a1[pl.ds(i, L)] = z
        a2[pl.ds(i, L)] = z

    for cp in cps:
        cp.wait()

    def edge(i, mask=None):
        s = ei_v[0, pl.ds(o16 + i, L)]
        d = ei_v[1, pl.ds(o16 + i, L)]
        plsc.addupdate_scatter(a0, [d], plsc.load_gather(g0, [s], mask=mask),
                               mask=mask)
        plsc.addupdate_scatter(a1, [d], plsc.load_gather(g1, [s], mask=mask),
                               mask=mask)
        plsc.addupdate_scatter(a2, [d], plsc.load_gather(g2, [s], mask=mask),
                               mask=mask)

    edge(0, mask=m_first)

    @pl.loop(L, EWF, step=L, unroll=4)
    def _(i):
        edge(i)

    edge(EWF, mask=m_last)

    for c, a in ((0, a0), (1, a1), (2, a2)):
        pltpu.sync_copy(a, stage.at[pl.ds(sid * NP, NP)])
        plsc.subcore_barrier()
        for cp in _fetch_partials(stage, buf_v, sem, sid * NSL, NS, NP):
            cp.wait()
        _reduce_partials(buf_v, sl_v, NS, 0)
        pltpu.sync_copy(
            sl_v,
            out_hbm.at[pl.ds(cid * (H * NP) + c * NP + sid * NSL, NSL)],
        )
        plsc.subcore_barrier()


def _tc0_body(wt_ref, x_ref, ht_ref):
    ht_ref[...] = lax.dot_general(wt_ref[...], x_ref[...],
                                  (((1,), (1,)), ((), ())),
                                  preferred_element_type=jnp.float32)


def _tc0(wt, x):
    return pl.pallas_call(
        _tc0_body,
        grid=(NP // BLK,),
        in_specs=[
            pl.BlockSpec((H, D), lambda i: (0, 0)),
            pl.BlockSpec((BLK, D), lambda i: (i, 0)),
        ],
        out_specs=pl.BlockSpec((H, BLK), lambda i: (0, i)),
        out_shape=jax.ShapeDtypeStruct((H, NP), jnp.float32),
    )(wt, x)


def _tc1_body(ht_ref, degp_ref, g_ref, dinv_ref):
    deg = degp_ref[0:1, :] + degp_ref[1:2, :] + 1.0
    dinv = lax.rsqrt(deg)
    g_ref[...] = ht_ref[...] * dinv
    dinv_ref[...] = dinv


def _tc1(ht, degp):
    return pl.pallas_call(
        _tc1_body,
        grid=(NP // BLK,),
        in_specs=[
            pl.BlockSpec((H, BLK), lambda i: (0, i)),
            pl.BlockSpec((NC, BLK), lambda i: (0, i)),
        ],
        out_specs=[
            pl.BlockSpec((H, BLK), lambda i: (0, i)),
            pl.BlockSpec((1, BLK), lambda i: (0, i)),
        ],
        out_shape=[
            jax.ShapeDtypeStruct((H, NP), jnp.float32),
            jax.ShapeDtypeStruct((1, NP), jnp.float32),
        ],
    )(ht, degp)


def _tc2_body(accp_ref, g_ref, dinv_ref, bg_ref, wo_ref, bo_ref,
              hr_ref, z_ref):
    a = accp_ref[...]
    acc = a[0:H, :] + a[H:2 * H, :] + g_ref[...]
    hg = acc * dinv_ref[...] + bg_ref[...]
    hr = jnp.maximum(hg, 0.0)
    hr_row = hr.T
    hr_ref[...] = hr_row
    z_ref[...] = jnp.dot(hr_row, wo_ref[...],
                         preferred_element_type=jnp.float32) + bo_ref[...]


def _tc2(accp, g, dinv, bg, wo, bo):
    return pl.pallas_call(
        _tc2_body,
        grid=(NP // BLK,),
        in_specs=[
            pl.BlockSpec((NC * H, BLK), lambda i: (0, i)),
            pl.BlockSpec((H, BLK), lambda i: (0, i)),
            pl.BlockSpec((1, BLK), lambda i: (0, i)),
            pl.BlockSpec((H, 1), lambda i: (0, 0)),
            pl.BlockSpec((H, C), lambda i: (0, 0)),
            pl.BlockSpec((1, C), lambda i: (0, 0)),
        ],
        out_specs=[
            pl.BlockSpec((BLK, H), lambda i: (i, 0)),
            pl.BlockSpec((BLK, C), lambda i: (i, 0)),
        ],
        out_shape=[
            jax.ShapeDtypeStruct((N, H), jnp.float32),
            jax.ShapeDtypeStruct((N, C), jnp.float32),
        ],
    )(accp, g, dinv, bg, wo, bo)


def kernel(x, edge_index, W_gcn, b_gcn, W_out, b_out):
    wt = W_gcn.T
    bg = b_gcn.reshape(H, 1)
    bo = b_out.reshape(1, C)

    ht = _tc0(wt, x)
    degp = _sc_degree(edge_index).reshape(NC, NP)
    g, dinv = _tc1(ht, degp)
    accp = _sc_aggregate(edge_index, g.reshape(H * NP))
    hr, z = _tc2(accp.reshape(NC * H, NP), g, dinv, bg, W_out, bo)
    return (hr, z)
